# baseline (speedup 1.0000x reference)
import functools

import jax
import jax.numpy as jnp
from jax import lax
from jax.experimental import pallas as pl
from jax.experimental.pallas import tpu as pltpu
from jax.experimental.pallas import tpu_sc as plsc

G = 129
GV = G - 1
S = 128
NRAYS = 4 * 2048
NW = 32
RPW = NRAYS // NW
NG = RPW // 16
XSTRIDE = GV * GV
YSTRIDE = GV


def _sc_build_table(dflat, cflat):
    mesh = plsc.VectorSubcoreMesh(core_axis_name="c", subcore_axis_name="s",
                                  num_cores=2, num_subcores=16)
    NROW = G * GV * GV

    @functools.partial(
        pl.kernel,
        mesh=mesh,
        compiler_params=pltpu.CompilerParams(
            needs_layout_passes=False, use_tc_tiling_on_sc=False),
        out_type=jax.ShapeDtypeStruct((NROW, 16), jnp.float32),
        scratch_types=[
            pltpu.VMEM((16656,), jnp.float32),
            pltpu.VMEM((49936,), jnp.float32),
            pltpu.VMEM((2, 512, 16), jnp.float32),
            pltpu.SemaphoreType.DMA,
        ],
    )
    def bk(d_hbm, c_hbm, table_hbm, dbuf, cbuf, obuf, sem):
        w = lax.axis_index("s") * 2 + lax.axis_index("c")
        iota = lax.iota(jnp.int32, 16)
        iota3 = iota * 3

        def build_chunk(par, rd, rc, ybase):
            pvec = jnp.full((16,), par, jnp.int32)
            for ys in range(4):
                db = rd + (ybase + ys) * G
                cb = rc + (ybase + ys) * (3 * G)
                for zg in range(8):
                    rowv = iota + (ys * 128 + zg * 16)
                    for q in range(4):
                        dy2, dz2 = q >> 1, q & 1
                        dcol = dbuf[pl.ds(db + dy2 * G + dz2 + zg * 16, 16)]
                        plsc.store_scatter(
                            obuf, [pvec, rowv, jnp.full((16,), 4 * q, jnp.int32)], dcol)
                        cbase = cb + (dy2 * G + dz2 + zg * 16) * 3
                        for ch in range(3):
                            col = plsc.load_gather(
                                cbuf, [iota3 + (cbase + ch)])
                            plsc.store_scatter(
                                obuf, [pvec, rowv,
                                       jnp.full((16,), 4 * q + 1 + ch, jnp.int32)], col)

        def drain():
            pltpu.make_async_copy(
                obuf.at[0], table_hbm.at[pl.ds(0, 512)], sem).wait()

        def per_x(xi, _):
            x = xi * 32 + w
            doff = x * (G * G)
            dbase8 = pl.multiple_of(lax.bitwise_and(doff, ~7), 8)
            rd = doff - dbase8
            pltpu.sync_copy(d_hbm.at[pl.ds(dbase8, 16656)], dbuf)
            coff = doff * 3
            cbase8 = pl.multiple_of(lax.bitwise_and(coff, ~7), 8)
            rc = coff - cbase8
            pltpu.sync_copy(c_hbm.at[pl.ds(cbase8, 49936)], cbuf)

            def per_yc(yc, _):
                @pl.when(yc >= 2)
                def _():
                    drain()
                par = lax.bitwise_and(yc, 1)
                build_chunk(par, rd, rc, yc * 4)
                rowbase = x * XSTRIDE + yc * 512
                pltpu.async_copy(obuf.at[par],
                                 table_hbm.at[pl.ds(rowbase, 512)], sem)
                return 0

            lax.fori_loop(0, 32, per_yc, 0)
            drain()
            drain()
            return 0

        lax.fori_loop(0, 4, per_x, 0)

        doff = 128 * (G * G) + (w * 4) * G
        dbase8 = pl.multiple_of(lax.bitwise_and(doff, ~7), 8)
        rd = doff - dbase8
        pltpu.sync_copy(d_hbm.at[pl.ds(dbase8, 656)], dbuf.at[pl.ds(0, 656)])
        coff = doff * 3
        cbase8 = pl.multiple_of(lax.bitwise_and(coff, ~7), 8)
        rc = coff - cbase8
        pltpu.sync_copy(c_hbm.at[pl.ds(cbase8, 1960)], cbuf.at[pl.ds(0, 1960)])
        build_chunk(0, rd, rc, 0)
        rowbase = 128 * XSTRIDE + w * 512
        pltpu.sync_copy(obuf.at[0], table_hbm.at[pl.ds(rowbase, 512)])

    return bk(dflat, cflat)


def _sc_render(table, o_r, d_r, g_r):
    mesh = plsc.VectorSubcoreMesh(core_axis_name="c", subcore_axis_name="s",
                                  num_cores=2, num_subcores=16)

    @functools.partial(
        pl.kernel,
        mesh=mesh,
        compiler_params=pltpu.CompilerParams(
            needs_layout_passes=False, use_tc_tiling_on_sc=False),
        out_type=(
            jax.ShapeDtypeStruct((NW, RPW), jnp.float32),
            jax.ShapeDtypeStruct((NW, 3, RPW), jnp.float32),
        ),
        scratch_types=[
            pltpu.VMEM((3, RPW), jnp.float32),
            pltpu.VMEM((3, RPW), jnp.float32),
            pltpu.VMEM((3, RPW), jnp.float32),
            pltpu.VMEM((32, S), jnp.int32),
            pltpu.VMEM((32 * S, 16), jnp.float32),
            pltpu.VMEM((3, S, 16), jnp.float32),
            pltpu.VMEM((RPW,), jnp.float32),
            pltpu.VMEM((3, RPW), jnp.float32),
            pltpu.SemaphoreType.DMA,
        ],
    )
    def k(table_hbm, o_hbm, d_hbm, g_hbm, depth_hbm, img_hbm,
          o_l, d_l, g_l, idx_buf, rows, frac, depth_l, img_l, sem):
        w = lax.axis_index("s") * 2 + lax.axis_index("c")
        pltpu.sync_copy(o_hbm.at[w], o_l)
        pltpu.sync_copy(d_hbm.at[w], d_l)
        pltpu.sync_copy(g_hbm.at[w], g_l)

        iota = lax.iota(jnp.int32, 16)

        def per_group(g, _):
            sl = pl.ds(g * 16, 16)
            ox = o_l[0, sl]
            oy = o_l[1, sl]
            oz = o_l[2, sl]
            dx = d_l[0, sl]
            dy = d_l[1, sl]
            dz = d_l[2, sl]
            ex = g_l[0, sl]
            ey = g_l[1, sl]
            ez = g_l[2, sl]

            hi = jnp.float32(GV)
            tn_x = (0.0 - ox) / ex
            tf_x = (hi - ox) / ex
            tn_y = (0.0 - oy) / ey
            tf_y = (hi - oy) / ey
            tn_z = (0.0 - oz) / ez
            tf_z = (hi - oz) / ez
            lo_x = jnp.where(tn_x < tf_x, tn_x, tf_x)
            hi_x = jnp.where(tn_x > tf_x, tn_x, tf_x)
            lo_y = jnp.where(tn_y < tf_y, tn_y, tf_y)
            hi_y = jnp.where(tn_y > tf_y, tn_y, tf_y)
            lo_z = jnp.where(tn_z < tf_z, tn_z, tf_z)
            hi_z = jnp.where(tn_z > tf_z, tn_z, tf_z)
            near0 = jnp.maximum(jnp.maximum(lo_x, lo_y), lo_z)
            far0 = jnp.minimum(jnp.minimum(hi_x, hi_y), hi_z)
            miss = far0 < near0
            near = jnp.where(miss, jnp.float32(1e9), near0)
            far = jnp.where(miss, jnp.float32(1e9), far0)
            near = jnp.maximum(near, jnp.float32(0.05))
            span = far - near
            delta = span * jnp.float32(1.0 / S)

            def p1(s, _):
                ts = s.astype(jnp.float32) * jnp.float32(1.0 / (S - 1))
                t = near + span * ts
                px = jnp.minimum(jnp.maximum(ox + dx * t, 0.0), hi)
                py = jnp.minimum(jnp.maximum(oy + dy * t, 0.0), hi)
                pz = jnp.minimum(jnp.maximum(oz + dz * t, 0.0), hi)
                x0 = jnp.minimum(px.astype(jnp.int32), GV - 1)
                y0 = jnp.minimum(py.astype(jnp.int32), GV - 1)
                z0 = jnp.minimum(pz.astype(jnp.int32), GV - 1)
                frac[0, s, :] = px - x0.astype(jnp.float32)
                frac[1, s, :] = py - y0.astype(jnp.float32)
                frac[2, s, :] = pz - z0.astype(jnp.float32)
                idx0 = x0 * XSTRIDE + y0 * YSTRIDE + z0
                j = lax.shift_right_logical(s, 2)
                off = lax.bitwise_and(s, 3) * 32
                idx_buf[j, pl.ds(off, 16)] = idx0
                idx_buf[j, pl.ds(off + 16, 16)] = idx0 + XSTRIDE
                return 0

            lax.fori_loop(0, S, p1, 0)

            handles = [
                pltpu.async_copy(table_hbm.at[idx_buf.at[j]],
                                 rows.at[pl.ds(j * S, S)], sem)
                for j in range(32)
            ]
            for h in handles:
                h.wait()

            def p3(s, carry):
                T, wsum, dep, ar, ag, ab = carry
                fx = frac[0, s, :]
                fy = frac[1, s, :]
                fz = frac[2, s, :]
                pbase = jnp.full((16,), s * 32, jnp.int32) + iota

                def blend_half(hbase):
                    rvec = pbase + hbase
                    v = [plsc.load_gather(rows, [rvec, jnp.full((16,), c, jnp.int32)])
                         for c in range(16)]
                    out = []
                    for ch in range(4):
                        a = v[ch] + fz * (v[4 + ch] - v[ch])
                        b2 = v[8 + ch] + fz * (v[12 + ch] - v[8 + ch])
                        out.append(a + fy * (b2 - a))
                    return out

                d0v, r0v, g0v, b0v = blend_half(0)
                d1v, r1v, g1v, b1v = blend_half(16)
                sig = d0v + fx * (d1v - d0v)
                sig = jnp.maximum(sig, 0.0)
                rr = r0v + fx * (r1v - r0v)
                gg = g0v + fx * (g1v - g0v)
                bb = b0v + fx * (b1v - b0v)
                rr = 1.0 / (1.0 + jnp.exp(-rr))
                gg = 1.0 / (1.0 + jnp.exp(-gg))
                bb = 1.0 / (1.0 + jnp.exp(-bb))

                alpha = 1.0 - jnp.exp(-sig * delta)
                wgt = alpha * T
                T = T * (1.0 - alpha + jnp.float32(1e-10))
                ts = s.astype(jnp.float32) * jnp.float32(1.0 / (S - 1))
                t = near + span * ts
                return (T, wsum + wgt, dep + wgt * t,
                        ar + wgt * rr, ag + wgt * gg, ab + wgt * bb)

            ones = jnp.full((16,), 1.0, jnp.float32)
            zeros = jnp.zeros((16,), jnp.float32)
            T, wsum, dep, ar, ag, ab = lax.fori_loop(
                0, S, p3, (ones, zeros, zeros, zeros, zeros, zeros))

            bg = 1.0 - wsum
            depth_l[sl] = (dep - near) / (far - near + jnp.float32(1e-8))
            img_l[0, sl] = ar + bg
            img_l[1, sl] = ag + bg
            img_l[2, sl] = ab + bg
            return 0

        lax.fori_loop(0, NG, per_group, 0)

        pltpu.sync_copy(depth_l, depth_hbm.at[w])
        pltpu.sync_copy(img_l, img_hbm.at[w])

    return k(table, o_r, d_r, g_r)


def kernel(rays_o, rays_d, exps, exp_ori, density_grid, color_grid, bound, num_steps):
    bnd = jnp.asarray(bound, dtype=jnp.float32)
    scale = jnp.float32(GV) / (2.0 * bnd)

    o = rays_o.reshape(NRAYS, 3)
    d = rays_d.reshape(NRAYS, 3)
    o_v = (o + bnd) * scale
    d_v = d * scale
    g_v = d_v + jnp.float32(1e-15) * scale

    def per_worker(x):
        return x.reshape(NW, RPW, 3).transpose(0, 2, 1)

    o_r = per_worker(o_v)
    d_r = per_worker(d_v)
    g_r = per_worker(g_v)

    dflat = jnp.concatenate(
        [density_grid.reshape(G * G * G), jnp.zeros((16,), jnp.float32)])
    cflat = jnp.concatenate(
        [color_grid.reshape(G * G * G * 3), jnp.zeros((32,), jnp.float32)])
    table = _sc_build_table(dflat, cflat)

    depth_w, img_w = _sc_render(table, o_r, d_r, g_r)

    depth = depth_w.reshape(4, 2048)
    image = img_w.transpose(0, 2, 1).reshape(4, 2048, 3)
    return depth, image, exps
